# Initial kernel scaffold; baseline (speedup 1.0000x reference)
import jax
import jax.numpy as jnp
from jax.experimental import pallas as pl


def kernel(feats, xyz):
    raise NotImplementedError("write your pallas kernel here")


# speedup vs baseline: 31.9079x; 31.9079x over previous
import functools

import jax
import jax.numpy as jnp
from jax import lax
from jax.experimental import pallas as pl
from jax.experimental.pallas import tpu as pltpu
from jax.experimental.pallas import tpu_sc as plsc

_B, _N, _M, _C = 16, 4096, 2048, 128


_G = 128


def _fps_body(x_ref, y_ref, z_ref, idx_ref, sx_ref, sy_ref, sz_ref, dist_ref):
    lane = lax.broadcasted_iota(jnp.int32, (_B, _N), 1)
    lane_g = lax.broadcasted_iota(jnp.int32, (_B, _G), 1)
    dist_ref[...] = jnp.full((_B, _N), 1e10, jnp.float32)

    def inner(j, st):
        f, ia, xa, ya, za = st
        x = x_ref[...]
        y = y_ref[...]
        z = z_ref[...]
        oh = lane == f
        cx = jnp.sum(jnp.where(oh, x, 0.0), axis=1, keepdims=True)
        cy = jnp.sum(jnp.where(oh, y, 0.0), axis=1, keepdims=True)
        cz = jnp.sum(jnp.where(oh, z, 0.0), axis=1, keepdims=True)
        mj = lane_g == j
        ia = jnp.where(mj, f, ia)
        xa = jnp.where(mj, cx, xa)
        ya = jnp.where(mj, cy, ya)
        za = jnp.where(mj, cz, za)
        dx = x - cx
        dy = y - cy
        dz = z - cz
        d = (dx * dx + dz * dz) + dy * dy
        nd = jnp.minimum(dist_ref[...], d)
        dist_ref[...] = nd
        mx = jnp.max(nd, axis=1, keepdims=True)
        fn = jnp.min(jnp.where(nd == mx, lane, _N), axis=1, keepdims=True)
        return (fn, ia, xa, ya, za)

    def outer(g, f):
        zi = jnp.zeros((_B, _G), jnp.int32)
        zf = jnp.zeros((_B, _G), jnp.float32)
        f, ia, xa, ya, za = lax.fori_loop(0, _G, inner, (f, zi, zf, zf, zf))
        base = pl.multiple_of(g * _G, _G)
        idx_ref[:, pl.ds(base, _G)] = ia
        sx_ref[:, pl.ds(base, _G)] = xa
        sy_ref[:, pl.ds(base, _G)] = ya
        sz_ref[:, pl.ds(base, _G)] = za
        return f

    lax.fori_loop(0, _M // _G, outer, jnp.zeros((_B, 1), jnp.int32))


def _fps(x, y, z):
    return pl.pallas_call(
        _fps_body,
        out_shape=(
            jax.ShapeDtypeStruct((_B, _M), jnp.int32),
            jax.ShapeDtypeStruct((_B, _M), jnp.float32),
            jax.ShapeDtypeStruct((_B, _M), jnp.float32),
            jax.ShapeDtypeStruct((_B, _M), jnp.float32),
        ),
        scratch_shapes=[pltpu.VMEM((_B, _N), jnp.float32)],
    )(x, y, z)


_NW = 32
_CW = _C // _NW


def _gather_body(feats_hbm, idx_hbm, out_hbm, idx_v, feat_v, out_v):
    wid = lax.axis_index("s") * 2 + lax.axis_index("c")
    c0 = wid * _CW
    for b in range(_B):
        pltpu.sync_copy(idx_hbm.at[b], idx_v)
        pltpu.sync_copy(feats_hbm.at[b, pl.ds(c0, _CW)], feat_v)
        for c in range(_CW):
            cvec = jnp.full((16,), c, jnp.int32)

            def inner(jj, carry):
                for u in range(4):
                    off = jj * 64 + u * 16
                    iv = idx_v[pl.ds(off, 16)]
                    out_v[c, pl.ds(off, 16)] = plsc.load_gather(
                        feat_v, [cvec, iv])
                return carry

            lax.fori_loop(0, _M // 64, inner, 0)
        pltpu.sync_copy(out_v, out_hbm.at[b, pl.ds(c0, _CW)])


def _gather(feats, idx):
    mesh = plsc.VectorSubcoreMesh(core_axis_name="c", subcore_axis_name="s")
    return pl.kernel(
        _gather_body,
        out_type=jax.ShapeDtypeStruct((_B, _C, _M), jnp.float32),
        mesh=mesh,
        compiler_params=pltpu.CompilerParams(needs_layout_passes=False),
        scratch_types=[
            pltpu.VMEM((_M,), jnp.int32),
            pltpu.VMEM((_CW, _N), jnp.float32),
            pltpu.VMEM((_CW, _M), jnp.float32),
        ],
    )(feats, idx)


def kernel(feats, xyz):
    xt = jnp.transpose(xyz, (2, 0, 1))
    idx, sx, sy, sz = _fps(xt[0], xt[1], xt[2])
    new_xyz = jnp.stack([sx, sy, sz], axis=-1)
    new_feats = _gather(feats, idx)
    return (new_feats, new_xyz)
